# baseline (speedup 1.0000x reference)
import functools

import jax
import jax.numpy as jnp
from jax import lax
from jax.experimental import pallas as pl
from jax.experimental.pallas import tpu as pltpu
from jax.experimental.pallas import tpu_sc as plsc

N = 10000
E = 160000
IN = 128
EDIM = 16
HID = 8
OUT = 64
HDIM = 64
NA = 8

NC = 2
NS = 16
NW = NC * NS
CHUNK = 128
NCHUNK = 2
EC = E // NCHUNK
TC_ = EC // CHUNK
MSGW = 128


@functools.lru_cache(maxsize=None)
def _sc_mesh():
    return plsc.VectorSubcoreMesh(
        core_axis_name="c", subcore_axis_name="s", num_cores=NC, num_subcores=NS)


def _gather_body(tbase, src1d, x_hbm, out, idx_v, rows_v, sem):
    cid = lax.axis_index("c")
    sid = lax.axis_index("s")
    wid = sid * NC + cid

    def body(j, carry):
        t = j * NW + wid

        @pl.when(t < TC_)
        def _():
            pltpu.sync_copy(src1d.at[pl.ds((tbase + t) * CHUNK, CHUNK)], idx_v)
            pltpu.async_copy(x_hbm.at[idx_v], rows_v, sem).wait()
            pltpu.sync_copy(rows_v, out.at[pl.ds(t * CHUNK, CHUNK)])

        return carry

    lax.fori_loop(0, -(-TC_ // NW), body, 0)


@functools.lru_cache(maxsize=None)
def _gather(k):
    return pl.kernel(
        functools.partial(_gather_body, k * TC_),
        out_type=jax.ShapeDtypeStruct((EC, IN), jnp.float32),
        mesh=_sc_mesh(),
        scratch_types=[
            pltpu.VMEM((CHUNK,), jnp.int32),
            pltpu.VMEM((CHUNK, IN), jnp.float32),
            pltpu.SemaphoreType.DMA,
        ],
    )


def _scatter_body(tbase, dst1d, msg_hbm, init_hbm, out, idx_v, rows_v, agg_sh, sem):
    cid = lax.axis_index("c")
    sid = lax.axis_index("s")
    wid = sid * NC + cid

    @pl.when(sid == 0)
    def _():
        pltpu.sync_copy(init_hbm.at[cid], agg_sh)

    plsc.subcore_barrier()

    def body(j, carry):
        t = j * NW + wid

        @pl.when(t < TC_)
        def _():
            pltpu.sync_copy(dst1d.at[pl.ds((tbase + t) * CHUNK, CHUNK)], idx_v)
            pltpu.sync_copy(msg_hbm.at[pl.ds(t * CHUNK, CHUNK)], rows_v)
            pltpu.sync_copy(rows_v, agg_sh.at[idx_v], add=True)

        return carry

    lax.fori_loop(0, -(-TC_ // NW), body, 0)
    plsc.subcore_barrier()

    @pl.when(sid == 0)
    def _():
        pltpu.sync_copy(agg_sh, out.at[cid])


@functools.lru_cache(maxsize=None)
def _scatter(k):
    return pl.kernel(
        functools.partial(_scatter_body, k * TC_),
        out_type=jax.ShapeDtypeStruct((NC, N, MSGW), jnp.float32),
        mesh=_sc_mesh(),
        scratch_types=[
            pltpu.VMEM((CHUNK,), jnp.int32),
            pltpu.VMEM((CHUNK, MSGW), jnp.float32),
            pltpu.VMEM_SHARED((N, MSGW), jnp.float32),
            pltpu.SemaphoreType.DMA,
        ],
    )


BE = 1000


def _edge_body(xs_ref, ea_ref, w_ref, b_ref, s_ref, out_ref):
    t = jnp.dot(ea_ref[...], w_ref[...],
                preferred_element_type=jnp.float32).astype(jnp.bfloat16)
    t = jnp.maximum(t + b_ref[...], jnp.bfloat16(0.0))
    xs = xs_ref[...].astype(jnp.bfloat16)
    u = jnp.concatenate([xs] * HID, axis=1) * t
    out_ref[...] = jnp.dot(u, s_ref[...], preferred_element_type=jnp.float32)


def _edge_msg(k, x_src, edge_attr, w1om, b1om, sel):
    koff = k * (EC // BE)
    return pl.pallas_call(
        _edge_body,
        grid=(EC // BE,),
        in_specs=[
            pl.BlockSpec((BE, IN), lambda i: (i, 0)),
            pl.BlockSpec((BE, EDIM), lambda i: (koff + i, 0)),
            pl.BlockSpec((EDIM, IN * HID), lambda i: (0, 0)),
            pl.BlockSpec((1, IN * HID), lambda i: (0, 0)),
            pl.BlockSpec((IN * HID, MSGW), lambda i: (0, 0)),
        ],
        out_specs=pl.BlockSpec((BE, MSGW), lambda i: (i, 0)),
        out_shape=jax.ShapeDtypeStruct((EC, MSGW), jnp.float32),
        compiler_params=pltpu.CompilerParams(
            dimension_semantics=("arbitrary",)),
    )(x_src, edge_attr, w1om, b1om, sel)


BN = 1000


def _node_body(x_ref, agg_ref, root_ref, bconv_ref, gamma_ref, beta_ref,
               wlin_ref, blin_ref, wq1_ref, bq1_ref, wq2_ref, bq2_ref, out_ref):
    h = (agg_ref[0, :, :HID] + agg_ref[1, :, :HID] + bconv_ref[...]
         + jnp.dot(x_ref[...], root_ref[...], preferred_element_type=jnp.float32))
    mu = jnp.mean(h, axis=1, keepdims=True)
    var = jnp.mean(jnp.square(h - mu), axis=1, keepdims=True)
    h = (h - mu) * lax.rsqrt(var + 1e-5) * gamma_ref[...] + beta_ref[...]
    h = jnp.maximum(h, 0.0)
    h = jnp.dot(h, wlin_ref[...], preferred_element_type=jnp.float32) + blin_ref[...]
    h = jnp.maximum(
        jnp.dot(h, wq1_ref[...], preferred_element_type=jnp.float32) + bq1_ref[...], 0.0)
    out_ref[...] = (jnp.dot(h, wq2_ref[...], preferred_element_type=jnp.float32)
                    + bq2_ref[...])


def _node_head(x, agg, rootw, bconv, gamma, beta, wlin, blin, wq1, bq1, wq2, bq2):
    full = lambda r, c: pl.BlockSpec((r, c), lambda i: (0, 0))
    return pl.pallas_call(
        _node_body,
        grid=(N // BN,),
        in_specs=[
            pl.BlockSpec((BN, IN), lambda i: (i, 0)),
            pl.BlockSpec((NC, BN, MSGW), lambda i: (0, i, 0)),
            full(IN, HID), full(1, HID), full(1, HID), full(1, HID),
            full(HID, OUT), full(1, OUT),
            full(OUT, HDIM), full(1, HDIM),
            full(HDIM, NA), full(1, NA),
        ],
        out_specs=pl.BlockSpec((BN, NA), lambda i: (i, 0)),
        out_shape=jax.ShapeDtypeStruct((N, NA), jnp.float32),
    )(x, agg, rootw, bconv, gamma, beta, wlin, blin, wq1, bq1, wq2, bq2)


def kernel(x, edge_index, edge_attr, W1, b1, root, bconv, gamma, beta,
           Wlin, blin, Wq1, bq1, Wq2, bq2):
    src1d = edge_index[0]
    dst1d = edge_index[1]

    w1om = W1.reshape(IN, HID, EDIM).transpose(1, 0, 2).reshape(IN * HID, EDIM).T
    b1om = b1.reshape(IN, HID).T.reshape(1, IN * HID)
    w1om = w1om.astype(jnp.bfloat16)
    b1om = b1om.astype(jnp.bfloat16)
    col = jnp.arange(IN * HID, dtype=jnp.int32) // IN
    sel = (col[:, None] == jnp.arange(MSGW, dtype=jnp.int32)[None, :])
    sel = sel.astype(jnp.bfloat16)

    agg = jnp.zeros((NC, N, MSGW), jnp.float32)
    x_srcs = [_gather(k)(src1d, x) for k in range(NCHUNK)]
    ea_bf = edge_attr.astype(jnp.bfloat16)
    msgs = [_edge_msg(k, x_srcs[k], ea_bf, w1om, b1om, sel)
            for k in range(NCHUNK)]
    for k in range(NCHUNK):
        agg = _scatter(k)(dst1d, msgs[k], agg)

    q = _node_head(x, agg, root, bconv.reshape(1, HID),
                   gamma.reshape(1, HID), beta.reshape(1, HID),
                   Wlin.T, blin.reshape(1, OUT), Wq1.T, bq1.reshape(1, HDIM),
                   Wq2.T, bq2.reshape(1, NA))
    return q


# speedup vs baseline: 1.0194x; 1.0194x over previous
import functools

import jax
import jax.numpy as jnp
from jax import lax
from jax.experimental import pallas as pl
from jax.experimental.pallas import tpu as pltpu
from jax.experimental.pallas import tpu_sc as plsc

N = 10000
E = 160000
IN = 128
EDIM = 16
HID = 8
OUT = 64
HDIM = 64
NA = 8

NC = 2
NS = 16
NW = NC * NS
CHUNK = 128
NCHUNK = 2
EC = E // NCHUNK
TC_ = EC // CHUNK
MSGW = 128


@functools.lru_cache(maxsize=None)
def _sc_mesh():
    return plsc.VectorSubcoreMesh(
        core_axis_name="c", subcore_axis_name="s", num_cores=NC, num_subcores=NS)


def _gather_body(tbase, src1d, x_hbm, out,
                 idx0, idx1, rows0, rows1, sem0, sem1):
    cid = lax.axis_index("c")
    sid = lax.axis_index("s")
    wid = sid * NC + cid
    idx = (idx0, idx1)
    rows = (rows0, rows1)
    sems = (sem0, sem1)

    def start(j, p):
        t = j * NW + wid

        @pl.when(t < TC_)
        def _():
            pltpu.sync_copy(src1d.at[pl.ds((tbase + t) * CHUNK, CHUNK)], idx[p])
            pltpu.async_copy(x_hbm.at[idx[p]], rows[p], sems[p])

    def finish(j, p):
        t = j * NW + wid

        @pl.when(t < TC_)
        def _():
            pltpu.make_async_copy(x_hbm.at[idx[p]], rows[p], sems[p]).wait()
            pltpu.sync_copy(rows[p], out.at[pl.ds(t * CHUNK, CHUNK)])

    nit = -(-TC_ // NW)
    start(0, 0)

    def body(j, carry):
        @pl.when(j % 2 == 0)
        def _():
            start(j + 1, 1)
            finish(j, 0)

        @pl.when(j % 2 == 1)
        def _():
            start(j + 1, 0)
            finish(j, 1)

        return carry

    lax.fori_loop(0, nit, body, 0)


@functools.lru_cache(maxsize=None)
def _gather(k):
    return pl.kernel(
        functools.partial(_gather_body, k * TC_),
        out_type=jax.ShapeDtypeStruct((EC, IN), jnp.float32),
        mesh=_sc_mesh(),
        scratch_types=[
            pltpu.VMEM((CHUNK,), jnp.int32),
            pltpu.VMEM((CHUNK,), jnp.int32),
            pltpu.VMEM((CHUNK, IN), jnp.float32),
            pltpu.VMEM((CHUNK, IN), jnp.float32),
            pltpu.SemaphoreType.DMA,
            pltpu.SemaphoreType.DMA,
        ],
    )


def _scatter_body(tbase, dst1d, msg_hbm, init_hbm, out, idx_v, rows_v, agg_sh, sem):
    cid = lax.axis_index("c")
    sid = lax.axis_index("s")
    wid = sid * NC + cid

    @pl.when(sid == 0)
    def _():
        pltpu.sync_copy(init_hbm.at[cid], agg_sh)

    plsc.subcore_barrier()

    def body(j, carry):
        t = j * NW + wid

        @pl.when(t < TC_)
        def _():
            pltpu.sync_copy(dst1d.at[pl.ds((tbase + t) * CHUNK, CHUNK)], idx_v)
            pltpu.sync_copy(msg_hbm.at[pl.ds(t * CHUNK, CHUNK)], rows_v)
            pltpu.sync_copy(rows_v, agg_sh.at[idx_v], add=True)

        return carry

    lax.fori_loop(0, -(-TC_ // NW), body, 0)
    plsc.subcore_barrier()

    @pl.when(sid == 0)
    def _():
        pltpu.sync_copy(agg_sh, out.at[cid])


@functools.lru_cache(maxsize=None)
def _scatter(k):
    return pl.kernel(
        functools.partial(_scatter_body, k * TC_),
        out_type=jax.ShapeDtypeStruct((NC, N, MSGW), jnp.float32),
        mesh=_sc_mesh(),
        scratch_types=[
            pltpu.VMEM((CHUNK,), jnp.int32),
            pltpu.VMEM((CHUNK, MSGW), jnp.float32),
            pltpu.VMEM_SHARED((N, MSGW), jnp.float32),
            pltpu.SemaphoreType.DMA,
        ],
    )


BE = 1000


def _edge_body(xs_ref, ea_ref, w_ref, b_ref, s_ref, out_ref):
    t = jnp.dot(ea_ref[...], w_ref[...],
                preferred_element_type=jnp.float32).astype(jnp.bfloat16)
    t = jnp.maximum(t + b_ref[...], jnp.bfloat16(0.0))
    xs = xs_ref[...].astype(jnp.bfloat16)
    u = jnp.concatenate([xs] * HID, axis=1) * t
    out_ref[...] = jnp.dot(u, s_ref[...], preferred_element_type=jnp.float32)


def _edge_msg(k, x_src, edge_attr, w1om, b1om, sel):
    koff = k * (EC // BE)
    return pl.pallas_call(
        _edge_body,
        grid=(EC // BE,),
        in_specs=[
            pl.BlockSpec((BE, IN), lambda i: (i, 0)),
            pl.BlockSpec((BE, EDIM), lambda i: (koff + i, 0)),
            pl.BlockSpec((EDIM, IN * HID), lambda i: (0, 0)),
            pl.BlockSpec((1, IN * HID), lambda i: (0, 0)),
            pl.BlockSpec((IN * HID, MSGW), lambda i: (0, 0)),
        ],
        out_specs=pl.BlockSpec((BE, MSGW), lambda i: (i, 0)),
        out_shape=jax.ShapeDtypeStruct((EC, MSGW), jnp.float32),
        compiler_params=pltpu.CompilerParams(
            dimension_semantics=("arbitrary",)),
    )(x_src, edge_attr, w1om, b1om, sel)


BN = 1000


def _node_body(x_ref, agg_ref, root_ref, bconv_ref, gamma_ref, beta_ref,
               wlin_ref, blin_ref, wq1_ref, bq1_ref, wq2_ref, bq2_ref, out_ref):
    h = (agg_ref[0, :, :HID] + agg_ref[1, :, :HID] + bconv_ref[...]
         + jnp.dot(x_ref[...], root_ref[...], preferred_element_type=jnp.float32))
    mu = jnp.mean(h, axis=1, keepdims=True)
    var = jnp.mean(jnp.square(h - mu), axis=1, keepdims=True)
    h = (h - mu) * lax.rsqrt(var + 1e-5) * gamma_ref[...] + beta_ref[...]
    h = jnp.maximum(h, 0.0)
    h = jnp.dot(h, wlin_ref[...], preferred_element_type=jnp.float32) + blin_ref[...]
    h = jnp.maximum(
        jnp.dot(h, wq1_ref[...], preferred_element_type=jnp.float32) + bq1_ref[...], 0.0)
    out_ref[...] = (jnp.dot(h, wq2_ref[...], preferred_element_type=jnp.float32)
                    + bq2_ref[...])


def _node_head(x, agg, rootw, bconv, gamma, beta, wlin, blin, wq1, bq1, wq2, bq2):
    full = lambda r, c: pl.BlockSpec((r, c), lambda i: (0, 0))
    return pl.pallas_call(
        _node_body,
        grid=(N // BN,),
        in_specs=[
            pl.BlockSpec((BN, IN), lambda i: (i, 0)),
            pl.BlockSpec((NC, BN, MSGW), lambda i: (0, i, 0)),
            full(IN, HID), full(1, HID), full(1, HID), full(1, HID),
            full(HID, OUT), full(1, OUT),
            full(OUT, HDIM), full(1, HDIM),
            full(HDIM, NA), full(1, NA),
        ],
        out_specs=pl.BlockSpec((BN, NA), lambda i: (i, 0)),
        out_shape=jax.ShapeDtypeStruct((N, NA), jnp.float32),
    )(x, agg, rootw, bconv, gamma, beta, wlin, blin, wq1, bq1, wq2, bq2)


def kernel(x, edge_index, edge_attr, W1, b1, root, bconv, gamma, beta,
           Wlin, blin, Wq1, bq1, Wq2, bq2):
    src1d = edge_index[0]
    dst1d = edge_index[1]

    w1om = W1.reshape(IN, HID, EDIM).transpose(1, 0, 2).reshape(IN * HID, EDIM).T
    b1om = b1.reshape(IN, HID).T.reshape(1, IN * HID)
    w1om = w1om.astype(jnp.bfloat16)
    b1om = b1om.astype(jnp.bfloat16)
    col = jnp.arange(IN * HID, dtype=jnp.int32) // IN
    sel = (col[:, None] == jnp.arange(MSGW, dtype=jnp.int32)[None, :])
    sel = sel.astype(jnp.bfloat16)

    agg = jnp.zeros((NC, N, MSGW), jnp.float32)
    x_srcs = [_gather(k)(src1d, x) for k in range(NCHUNK)]
    ea_bf = edge_attr.astype(jnp.bfloat16)
    msgs = [_edge_msg(k, x_srcs[k], ea_bf, w1om, b1om, sel)
            for k in range(NCHUNK)]
    for k in range(NCHUNK):
        agg = _scatter(k)(dst1d, msgs[k], agg)

    q = _node_head(x, agg, root, bconv.reshape(1, HID),
                   gamma.reshape(1, HID), beta.reshape(1, HID),
                   Wlin.T, blin.reshape(1, OUT), Wq1.T, bq1.reshape(1, HDIM),
                   Wq2.T, bq2.reshape(1, NA))
    return q


# speedup vs baseline: 1.0767x; 1.0562x over previous
import functools

import jax
import jax.numpy as jnp
from jax import lax
from jax.experimental import pallas as pl
from jax.experimental.pallas import tpu as pltpu
from jax.experimental.pallas import tpu_sc as plsc

N = 10000
E = 160000
IN = 128
EDIM = 16
HID = 8
OUT = 64
HDIM = 64
NA = 8

NC = 2
NS = 16
NW = NC * NS
CHUNK = 128
NCHUNK = 2
EC = E // NCHUNK
TC_ = EC // CHUNK
MSGW = 128


@functools.lru_cache(maxsize=None)
def _sc_mesh():
    return plsc.VectorSubcoreMesh(
        core_axis_name="c", subcore_axis_name="s", num_cores=NC, num_subcores=NS)


def _gather_body(tbase, src1d, x_hbm, out,
                 idx0, idx1, rows0, rows1, sem0, sem1):
    cid = lax.axis_index("c")
    sid = lax.axis_index("s")
    wid = sid * NC + cid
    idx = (idx0, idx1)
    rows = (rows0, rows1)
    sems = (sem0, sem1)

    def start(j, p):
        t = j * NW + wid

        @pl.when(t < TC_)
        def _():
            pltpu.sync_copy(src1d.at[pl.ds((tbase + t) * CHUNK, CHUNK)], idx[p])
            pltpu.async_copy(x_hbm.at[idx[p]], rows[p], sems[p])

    def finish(j, p):
        t = j * NW + wid

        @pl.when(t < TC_)
        def _():
            pltpu.make_async_copy(x_hbm.at[idx[p]], rows[p], sems[p]).wait()
            pltpu.sync_copy(rows[p], out.at[pl.ds(t * CHUNK, CHUNK)])

    nit = -(-TC_ // NW)
    start(0, 0)

    def body(j, carry):
        @pl.when(j % 2 == 0)
        def _():
            start(j + 1, 1)
            finish(j, 0)

        @pl.when(j % 2 == 1)
        def _():
            start(j + 1, 0)
            finish(j, 1)

        return carry

    lax.fori_loop(0, nit, body, 0)


@functools.lru_cache(maxsize=None)
def _gather(k):
    return pl.kernel(
        functools.partial(_gather_body, k * TC_),
        out_type=jax.ShapeDtypeStruct((EC, IN), jnp.float32),
        mesh=_sc_mesh(),
        scratch_types=[
            pltpu.VMEM((CHUNK,), jnp.int32),
            pltpu.VMEM((CHUNK,), jnp.int32),
            pltpu.VMEM((CHUNK, IN), jnp.float32),
            pltpu.VMEM((CHUNK, IN), jnp.float32),
            pltpu.SemaphoreType.DMA,
            pltpu.SemaphoreType.DMA,
        ],
    )


def _scatter_body(tbase, dst1d, msg_hbm, init_hbm, out,
                  idx0, idx1, rows0, rows1, agg_sh, sem0, sem1):
    cid = lax.axis_index("c")
    sid = lax.axis_index("s")
    wid = sid * NC + cid
    idx = (idx0, idx1)
    rows = (rows0, rows1)
    sems = (sem0, sem1)

    @pl.when(sid == 0)
    def _():
        pltpu.sync_copy(init_hbm.at[cid], agg_sh)

    plsc.subcore_barrier()

    def start(j, p):
        t = j * NW + wid

        @pl.when(t < TC_)
        def _():
            pltpu.sync_copy(dst1d.at[pl.ds((tbase + t) * CHUNK, CHUNK)], idx[p])
            pltpu.async_copy(msg_hbm.at[pl.ds(t * CHUNK, CHUNK)], rows[p], sems[p])

    def finish(j, p):
        t = j * NW + wid

        @pl.when(t < TC_)
        def _():
            pltpu.make_async_copy(
                msg_hbm.at[pl.ds(t * CHUNK, CHUNK)], rows[p], sems[p]).wait()
            pltpu.sync_copy(rows[p], agg_sh.at[idx[p]], add=True)

    nit = -(-TC_ // NW)
    start(0, 0)

    def body(j, carry):
        @pl.when(j % 2 == 0)
        def _():
            start(j + 1, 1)
            finish(j, 0)

        @pl.when(j % 2 == 1)
        def _():
            start(j + 1, 0)
            finish(j, 1)

        return carry

    lax.fori_loop(0, nit, body, 0)
    plsc.subcore_barrier()

    @pl.when(sid == 0)
    def _():
        pltpu.sync_copy(agg_sh, out.at[cid])


@functools.lru_cache(maxsize=None)
def _scatter(k):
    return pl.kernel(
        functools.partial(_scatter_body, k * TC_),
        out_type=jax.ShapeDtypeStruct((NC, N, MSGW), jnp.float32),
        mesh=_sc_mesh(),
        scratch_types=[
            pltpu.VMEM((CHUNK,), jnp.int32),
            pltpu.VMEM((CHUNK,), jnp.int32),
            pltpu.VMEM((CHUNK, MSGW), jnp.float32),
            pltpu.VMEM((CHUNK, MSGW), jnp.float32),
            pltpu.VMEM_SHARED((N, MSGW), jnp.float32),
            pltpu.SemaphoreType.DMA,
            pltpu.SemaphoreType.DMA,
        ],
    )


BE = 1000


def _edge_body(xs_ref, ea_ref, w_ref, b_ref, s_ref, out_ref):
    t = jnp.dot(ea_ref[...], w_ref[...],
                preferred_element_type=jnp.float32).astype(jnp.bfloat16)
    t = jnp.maximum(t + b_ref[...], jnp.bfloat16(0.0))
    xs = xs_ref[...].astype(jnp.bfloat16)
    u = jnp.concatenate([xs] * HID, axis=1) * t
    out_ref[...] = jnp.dot(u, s_ref[...], preferred_element_type=jnp.float32)


def _edge_msg(k, x_src, edge_attr, w1om, b1om, sel):
    koff = k * (EC // BE)
    return pl.pallas_call(
        _edge_body,
        grid=(EC // BE,),
        in_specs=[
            pl.BlockSpec((BE, IN), lambda i: (i, 0)),
            pl.BlockSpec((BE, EDIM), lambda i: (koff + i, 0)),
            pl.BlockSpec((EDIM, IN * HID), lambda i: (0, 0)),
            pl.BlockSpec((1, IN * HID), lambda i: (0, 0)),
            pl.BlockSpec((IN * HID, MSGW), lambda i: (0, 0)),
        ],
        out_specs=pl.BlockSpec((BE, MSGW), lambda i: (i, 0)),
        out_shape=jax.ShapeDtypeStruct((EC, MSGW), jnp.float32),
        compiler_params=pltpu.CompilerParams(
            dimension_semantics=("arbitrary",)),
    )(x_src, edge_attr, w1om, b1om, sel)


BN = 1000


def _node_body(x_ref, agg_ref, root_ref, bconv_ref, gamma_ref, beta_ref,
               wlin_ref, blin_ref, wq1_ref, bq1_ref, wq2_ref, bq2_ref, out_ref):
    h = (agg_ref[0, :, :HID] + agg_ref[1, :, :HID] + bconv_ref[...]
         + jnp.dot(x_ref[...], root_ref[...], preferred_element_type=jnp.float32))
    mu = jnp.mean(h, axis=1, keepdims=True)
    var = jnp.mean(jnp.square(h - mu), axis=1, keepdims=True)
    h = (h - mu) * lax.rsqrt(var + 1e-5) * gamma_ref[...] + beta_ref[...]
    h = jnp.maximum(h, 0.0)
    h = jnp.dot(h, wlin_ref[...], preferred_element_type=jnp.float32) + blin_ref[...]
    h = jnp.maximum(
        jnp.dot(h, wq1_ref[...], preferred_element_type=jnp.float32) + bq1_ref[...], 0.0)
    out_ref[...] = (jnp.dot(h, wq2_ref[...], preferred_element_type=jnp.float32)
                    + bq2_ref[...])


def _node_head(x, agg, rootw, bconv, gamma, beta, wlin, blin, wq1, bq1, wq2, bq2):
    full = lambda r, c: pl.BlockSpec((r, c), lambda i: (0, 0))
    return pl.pallas_call(
        _node_body,
        grid=(N // BN,),
        in_specs=[
            pl.BlockSpec((BN, IN), lambda i: (i, 0)),
            pl.BlockSpec((NC, BN, MSGW), lambda i: (0, i, 0)),
            full(IN, HID), full(1, HID), full(1, HID), full(1, HID),
            full(HID, OUT), full(1, OUT),
            full(OUT, HDIM), full(1, HDIM),
            full(HDIM, NA), full(1, NA),
        ],
        out_specs=pl.BlockSpec((BN, NA), lambda i: (i, 0)),
        out_shape=jax.ShapeDtypeStruct((N, NA), jnp.float32),
    )(x, agg, rootw, bconv, gamma, beta, wlin, blin, wq1, bq1, wq2, bq2)


def kernel(x, edge_index, edge_attr, W1, b1, root, bconv, gamma, beta,
           Wlin, blin, Wq1, bq1, Wq2, bq2):
    src1d = edge_index[0]
    dst1d = edge_index[1]

    w1om = W1.reshape(IN, HID, EDIM).transpose(1, 0, 2).reshape(IN * HID, EDIM).T
    b1om = b1.reshape(IN, HID).T.reshape(1, IN * HID)
    w1om = w1om.astype(jnp.bfloat16)
    b1om = b1om.astype(jnp.bfloat16)
    col = jnp.arange(IN * HID, dtype=jnp.int32) // IN
    sel = (col[:, None] == jnp.arange(MSGW, dtype=jnp.int32)[None, :])
    sel = sel.astype(jnp.bfloat16)

    agg = jnp.zeros((NC, N, MSGW), jnp.float32)
    x_srcs = [_gather(k)(src1d, x) for k in range(NCHUNK)]
    ea_bf = edge_attr.astype(jnp.bfloat16)
    msgs = [_edge_msg(k, x_srcs[k], ea_bf, w1om, b1om, sel)
            for k in range(NCHUNK)]
    for k in range(NCHUNK):
        agg = _scatter(k)(dst1d, msgs[k], agg)

    q = _node_head(x, agg, root, bconv.reshape(1, HID),
                   gamma.reshape(1, HID), beta.reshape(1, HID),
                   Wlin.T, blin.reshape(1, OUT), Wq1.T, bq1.reshape(1, HDIM),
                   Wq2.T, bq2.reshape(1, NA))
    return q


# speedup vs baseline: 1.1030x; 1.0244x over previous
import functools

import jax
import jax.numpy as jnp
from jax import lax
from jax.experimental import pallas as pl
from jax.experimental.pallas import tpu as pltpu
from jax.experimental.pallas import tpu_sc as plsc

N = 10000
E = 160000
IN = 128
EDIM = 16
HID = 8
OUT = 64
HDIM = 64
NA = 8

NC = 2
NS = 16
NW = NC * NS
CHUNK = 128
NCHUNK = 2
EC = E // NCHUNK
TC_ = EC // CHUNK
MSGW = 128


@functools.lru_cache(maxsize=None)
def _sc_mesh():
    return plsc.VectorSubcoreMesh(
        core_axis_name="c", subcore_axis_name="s", num_cores=NC, num_subcores=NS)


def _gather_body(tbase, ei_hbm, x_hbm, out,
                 idx0, idx1, rows0, rows1, sem0, sem1):
    cid = lax.axis_index("c")
    sid = lax.axis_index("s")
    wid = sid * NC + cid
    idx = (idx0, idx1)
    rows = (rows0, rows1)
    sems = (sem0, sem1)

    def start(j, p):
        t = j * NW + wid

        @pl.when(t < TC_)
        def _():
            pltpu.sync_copy(ei_hbm.at[0, pl.ds((tbase + t) * CHUNK, CHUNK)], idx[p])
            pltpu.async_copy(x_hbm.at[idx[p]], rows[p], sems[p])

    def finish(j, p):
        t = j * NW + wid

        @pl.when(t < TC_)
        def _():
            pltpu.make_async_copy(x_hbm.at[idx[p]], rows[p], sems[p]).wait()
            pltpu.sync_copy(rows[p], out.at[pl.ds(t * CHUNK, CHUNK)])

    nit = -(-TC_ // NW)
    start(0, 0)

    def body(j, carry):
        @pl.when(j % 2 == 0)
        def _():
            start(j + 1, 1)
            finish(j, 0)

        @pl.when(j % 2 == 1)
        def _():
            start(j + 1, 0)
            finish(j, 1)

        return carry

    lax.fori_loop(0, nit, body, 0)


@functools.lru_cache(maxsize=None)
def _gather(k):
    return pl.kernel(
        functools.partial(_gather_body, k * TC_),
        out_type=jax.ShapeDtypeStruct((EC, IN), jnp.float32),
        mesh=_sc_mesh(),
        scratch_types=[
            pltpu.VMEM((CHUNK,), jnp.int32),
            pltpu.VMEM((CHUNK,), jnp.int32),
            pltpu.VMEM((CHUNK, IN), jnp.float32),
            pltpu.VMEM((CHUNK, IN), jnp.float32),
            pltpu.SemaphoreType.DMA,
            pltpu.SemaphoreType.DMA,
        ],
    )


def _scatter_body(tbase, ei_hbm, msg_hbm, init_hbm, out,
                  idx0, idx1, rows0, rows1, agg_sh, sem0, sem1):
    cid = lax.axis_index("c")
    sid = lax.axis_index("s")
    wid = sid * NC + cid
    idx = (idx0, idx1)
    rows = (rows0, rows1)
    sems = (sem0, sem1)

    @pl.when(sid == 0)
    def _():
        pltpu.sync_copy(init_hbm.at[cid], agg_sh)

    plsc.subcore_barrier()

    def start(j, p):
        t = j * NW + wid

        @pl.when(t < TC_)
        def _():
            pltpu.sync_copy(ei_hbm.at[1, pl.ds((tbase + t) * CHUNK, CHUNK)], idx[p])
            pltpu.async_copy(msg_hbm.at[pl.ds(t * CHUNK, CHUNK)], rows[p], sems[p])

    def finish(j, p):
        t = j * NW + wid

        @pl.when(t < TC_)
        def _():
            pltpu.make_async_copy(
                msg_hbm.at[pl.ds(t * CHUNK, CHUNK)], rows[p], sems[p]).wait()
            pltpu.sync_copy(rows[p], agg_sh.at[idx[p]], add=True)

    nit = -(-TC_ // NW)
    start(0, 0)

    def body(j, carry):
        @pl.when(j % 2 == 0)
        def _():
            start(j + 1, 1)
            finish(j, 0)

        @pl.when(j % 2 == 1)
        def _():
            start(j + 1, 0)
            finish(j, 1)

        return carry

    lax.fori_loop(0, nit, body, 0)
    plsc.subcore_barrier()

    @pl.when(sid == 0)
    def _():
        pltpu.sync_copy(agg_sh, out.at[cid])


@functools.lru_cache(maxsize=None)
def _scatter(k):
    return pl.kernel(
        functools.partial(_scatter_body, k * TC_),
        out_type=jax.ShapeDtypeStruct((NC, N, MSGW), jnp.float32),
        mesh=_sc_mesh(),
        scratch_types=[
            pltpu.VMEM((CHUNK,), jnp.int32),
            pltpu.VMEM((CHUNK,), jnp.int32),
            pltpu.VMEM((CHUNK, MSGW), jnp.float32),
            pltpu.VMEM((CHUNK, MSGW), jnp.float32),
            pltpu.VMEM_SHARED((N, MSGW), jnp.float32),
            pltpu.SemaphoreType.DMA,
            pltpu.SemaphoreType.DMA,
        ],
    )


BE = 1000


def _edge_body(xs_ref, ea_ref, w_ref, b_ref, s_ref, out_ref):
    t = jnp.dot(ea_ref[...], w_ref[...],
                preferred_element_type=jnp.float32).astype(jnp.bfloat16)
    t = jnp.maximum(t + b_ref[...], jnp.bfloat16(0.0))
    xs = xs_ref[...].astype(jnp.bfloat16)
    u = jnp.concatenate([xs] * HID, axis=1) * t
    out_ref[...] = jnp.dot(u, s_ref[...], preferred_element_type=jnp.float32)


def _edge_msg(k, x_src, edge_attr, w1om, b1om, sel):
    koff = k * (EC // BE)
    return pl.pallas_call(
        _edge_body,
        grid=(EC // BE,),
        in_specs=[
            pl.BlockSpec((BE, IN), lambda i: (i, 0)),
            pl.BlockSpec((BE, EDIM), lambda i: (koff + i, 0)),
            pl.BlockSpec((EDIM, IN * HID), lambda i: (0, 0)),
            pl.BlockSpec((1, IN * HID), lambda i: (0, 0)),
            pl.BlockSpec((IN * HID, MSGW), lambda i: (0, 0)),
        ],
        out_specs=pl.BlockSpec((BE, MSGW), lambda i: (i, 0)),
        out_shape=jax.ShapeDtypeStruct((EC, MSGW), jnp.float32),
        compiler_params=pltpu.CompilerParams(
            dimension_semantics=("arbitrary",)),
    )(x_src, edge_attr, w1om, b1om, sel)


BN = 1000


def _node_body(x_ref, agg_ref, root_ref, bconv_ref, gamma_ref, beta_ref,
               wlin_ref, blin_ref, wq1_ref, bq1_ref, wq2_ref, bq2_ref, out_ref):
    h = (agg_ref[0, :, :HID] + agg_ref[1, :, :HID] + bconv_ref[...]
         + jnp.dot(x_ref[...], root_ref[...], preferred_element_type=jnp.float32))
    mu = jnp.mean(h, axis=1, keepdims=True)
    var = jnp.mean(jnp.square(h - mu), axis=1, keepdims=True)
    h = (h - mu) * lax.rsqrt(var + 1e-5) * gamma_ref[...] + beta_ref[...]
    h = jnp.maximum(h, 0.0)
    h = jnp.dot(h, wlin_ref[...], preferred_element_type=jnp.float32) + blin_ref[...]
    h = jnp.maximum(
        jnp.dot(h, wq1_ref[...], preferred_element_type=jnp.float32) + bq1_ref[...], 0.0)
    out_ref[...] = (jnp.dot(h, wq2_ref[...], preferred_element_type=jnp.float32)
                    + bq2_ref[...])


def _node_head(x, agg, rootw, bconv, gamma, beta, wlin, blin, wq1, bq1, wq2, bq2):
    full = lambda r, c: pl.BlockSpec((r, c), lambda i: (0, 0))
    return pl.pallas_call(
        _node_body,
        grid=(N // BN,),
        in_specs=[
            pl.BlockSpec((BN, IN), lambda i: (i, 0)),
            pl.BlockSpec((NC, BN, MSGW), lambda i: (0, i, 0)),
            full(IN, HID), full(1, HID), full(1, HID), full(1, HID),
            full(HID, OUT), full(1, OUT),
            full(OUT, HDIM), full(1, HDIM),
            full(HDIM, NA), full(1, NA),
        ],
        out_specs=pl.BlockSpec((BN, NA), lambda i: (i, 0)),
        out_shape=jax.ShapeDtypeStruct((N, NA), jnp.float32),
    )(x, agg, rootw, bconv, gamma, beta, wlin, blin, wq1, bq1, wq2, bq2)


def kernel(x, edge_index, edge_attr, W1, b1, root, bconv, gamma, beta,
           Wlin, blin, Wq1, bq1, Wq2, bq2):

    w1om = W1.reshape(IN, HID, EDIM).transpose(1, 0, 2).reshape(IN * HID, EDIM).T
    b1om = b1.reshape(IN, HID).T.reshape(1, IN * HID)
    w1om = w1om.astype(jnp.bfloat16)
    b1om = b1om.astype(jnp.bfloat16)
    col = jnp.arange(IN * HID, dtype=jnp.int32) // IN
    sel = (col[:, None] == jnp.arange(MSGW, dtype=jnp.int32)[None, :])
    sel = sel.astype(jnp.bfloat16)

    agg = jnp.zeros((NC, N, MSGW), jnp.float32)
    x_srcs = [_gather(k)(edge_index, x) for k in range(NCHUNK)]
    ea_bf = edge_attr.astype(jnp.bfloat16)
    msgs = [_edge_msg(k, x_srcs[k], ea_bf, w1om, b1om, sel)
            for k in range(NCHUNK)]
    for k in range(NCHUNK):
        agg = _scatter(k)(edge_index, msgs[k], agg)

    q = _node_head(x, agg, root, bconv.reshape(1, HID),
                   gamma.reshape(1, HID), beta.reshape(1, HID),
                   Wlin.T, blin.reshape(1, OUT), Wq1.T, bq1.reshape(1, HDIM),
                   Wq2.T, bq2.reshape(1, NA))
    return q


# speedup vs baseline: 1.2123x; 1.0991x over previous
import functools

import jax
import jax.numpy as jnp
from jax import lax
from jax.experimental import pallas as pl
from jax.experimental.pallas import tpu as pltpu
from jax.experimental.pallas import tpu_sc as plsc

N = 10000
E = 160000
IN = 128
EDIM = 16
HID = 8
OUT = 64
HDIM = 64
NA = 8

NC = 2
NS = 16
NW = NC * NS
CHUNK = 128
NCHUNK = 2
EC = E // NCHUNK
TC_ = EC // CHUNK
MSGW = 128


@functools.lru_cache(maxsize=None)
def _sc_mesh():
    return plsc.VectorSubcoreMesh(
        core_axis_name="c", subcore_axis_name="s", num_cores=NC, num_subcores=NS)


def _gather_body(tbase, ei_hbm, x_hbm, out,
                 idx0, idx1, rows0, rows1, sem0, sem1):
    cid = lax.axis_index("c")
    sid = lax.axis_index("s")
    wid = sid * NC + cid
    idx = (idx0, idx1)
    rows = (rows0, rows1)
    sems = (sem0, sem1)

    def start(j, p):
        t = j * NW + wid

        @pl.when(t < TC_)
        def _():
            pltpu.sync_copy(ei_hbm.at[0, pl.ds((tbase + t) * CHUNK, CHUNK)], idx[p])
            pltpu.async_copy(x_hbm.at[idx[p]], rows[p], sems[p])

    def finish(j, p):
        t = j * NW + wid

        @pl.when(t < TC_)
        def _():
            pltpu.make_async_copy(x_hbm.at[idx[p]], rows[p], sems[p]).wait()
            pltpu.sync_copy(rows[p], out.at[pl.ds(t * CHUNK, CHUNK)])

    nit = -(-TC_ // NW)
    start(0, 0)

    def body(j, carry):
        @pl.when(j % 2 == 0)
        def _():
            start(j + 1, 1)
            finish(j, 0)

        @pl.when(j % 2 == 1)
        def _():
            start(j + 1, 0)
            finish(j, 1)

        return carry

    lax.fori_loop(0, nit, body, 0)


@functools.lru_cache(maxsize=None)
def _gather(k):
    return pl.kernel(
        functools.partial(_gather_body, k * TC_),
        out_type=jax.ShapeDtypeStruct((EC, IN), jnp.float32),
        mesh=_sc_mesh(),
        scratch_types=[
            pltpu.VMEM((CHUNK,), jnp.int32),
            pltpu.VMEM((CHUNK,), jnp.int32),
            pltpu.VMEM((CHUNK, IN), jnp.float32),
            pltpu.VMEM((CHUNK, IN), jnp.float32),
            pltpu.SemaphoreType.DMA,
            pltpu.SemaphoreType.DMA,
        ],
    )


def _scatter_body(tbase, ei_hbm, msg_hbm, init_hbm, out,
                  idx0, idx1, rows0, rows1, agg_sh, sem0, sem1):
    cid = lax.axis_index("c")
    sid = lax.axis_index("s")
    wid = sid * NC + cid
    idx = (idx0, idx1)
    rows = (rows0, rows1)
    sems = (sem0, sem1)

    @pl.when(sid == 0)
    def _():
        pltpu.sync_copy(init_hbm.at[cid], agg_sh)

    plsc.subcore_barrier()

    def start(j, p):
        t = j * NW + wid

        @pl.when(t < TC_)
        def _():
            pltpu.sync_copy(ei_hbm.at[1, pl.ds((tbase + t) * CHUNK, CHUNK)], idx[p])
            pltpu.async_copy(msg_hbm.at[pl.ds(t * CHUNK, CHUNK)], rows[p], sems[p])

    def finish(j, p):
        t = j * NW + wid

        @pl.when(t < TC_)
        def _():
            pltpu.make_async_copy(
                msg_hbm.at[pl.ds(t * CHUNK, CHUNK)], rows[p], sems[p]).wait()
            pltpu.sync_copy(rows[p], agg_sh.at[idx[p]], add=True)

    nit = -(-TC_ // NW)
    start(0, 0)

    def body(j, carry):
        @pl.when(j % 2 == 0)
        def _():
            start(j + 1, 1)
            finish(j, 0)

        @pl.when(j % 2 == 1)
        def _():
            start(j + 1, 0)
            finish(j, 1)

        return carry

    lax.fori_loop(0, nit, body, 0)
    plsc.subcore_barrier()

    @pl.when(sid == 0)
    def _():
        pltpu.sync_copy(agg_sh, out.at[cid])


@functools.lru_cache(maxsize=None)
def _scatter(k):
    return pl.kernel(
        functools.partial(_scatter_body, k * TC_),
        out_type=jax.ShapeDtypeStruct((NC, N, MSGW), jnp.float32),
        mesh=_sc_mesh(),
        scratch_types=[
            pltpu.VMEM((CHUNK,), jnp.int32),
            pltpu.VMEM((CHUNK,), jnp.int32),
            pltpu.VMEM((CHUNK, MSGW), jnp.float32),
            pltpu.VMEM((CHUNK, MSGW), jnp.float32),
            pltpu.VMEM_SHARED((N, MSGW), jnp.float32),
            pltpu.SemaphoreType.DMA,
            pltpu.SemaphoreType.DMA,
        ],
    )


BE = 2000


def _edge_body(xs_ref, ea_ref, w_ref, b_ref, s_ref, out_ref):
    t = jnp.dot(ea_ref[...], w_ref[...],
                preferred_element_type=jnp.float32).astype(jnp.bfloat16)
    t = jnp.maximum(t + b_ref[...], jnp.bfloat16(0.0))
    xs = xs_ref[...].astype(jnp.bfloat16)
    u = jnp.concatenate([xs] * HID, axis=1) * t
    out_ref[...] = jnp.dot(u, s_ref[...], preferred_element_type=jnp.float32)


def _edge_msg(k, x_src, edge_attr, w1om, b1om, sel):
    koff = k * (EC // BE)
    return pl.pallas_call(
        _edge_body,
        grid=(EC // BE,),
        in_specs=[
            pl.BlockSpec((BE, IN), lambda i: (i, 0)),
            pl.BlockSpec((BE, EDIM), lambda i: (koff + i, 0)),
            pl.BlockSpec((EDIM, IN * HID), lambda i: (0, 0)),
            pl.BlockSpec((1, IN * HID), lambda i: (0, 0)),
            pl.BlockSpec((IN * HID, MSGW), lambda i: (0, 0)),
        ],
        out_specs=pl.BlockSpec((BE, MSGW), lambda i: (i, 0)),
        out_shape=jax.ShapeDtypeStruct((EC, MSGW), jnp.float32),
        compiler_params=pltpu.CompilerParams(
            dimension_semantics=("arbitrary",)),
    )(x_src, edge_attr, w1om, b1om, sel)


BN = 1000


def _node_body(x_ref, agg_ref, root_ref, bconv_ref, gamma_ref, beta_ref,
               wlin_ref, blin_ref, wq1_ref, bq1_ref, wq2_ref, bq2_ref, out_ref):
    h = (agg_ref[0, :, :HID] + agg_ref[1, :, :HID] + bconv_ref[...]
         + jnp.dot(x_ref[...], root_ref[...], preferred_element_type=jnp.float32))
    mu = jnp.mean(h, axis=1, keepdims=True)
    var = jnp.mean(jnp.square(h - mu), axis=1, keepdims=True)
    h = (h - mu) * lax.rsqrt(var + 1e-5) * gamma_ref[...] + beta_ref[...]
    h = jnp.maximum(h, 0.0)
    h = jnp.dot(h, wlin_ref[...], preferred_element_type=jnp.float32) + blin_ref[...]
    h = jnp.maximum(
        jnp.dot(h, wq1_ref[...], preferred_element_type=jnp.float32) + bq1_ref[...], 0.0)
    out_ref[...] = (jnp.dot(h, wq2_ref[...], preferred_element_type=jnp.float32)
                    + bq2_ref[...])


def _node_head(x, agg, rootw, bconv, gamma, beta, wlin, blin, wq1, bq1, wq2, bq2):
    full = lambda r, c: pl.BlockSpec((r, c), lambda i: (0, 0))
    return pl.pallas_call(
        _node_body,
        grid=(N // BN,),
        in_specs=[
            pl.BlockSpec((BN, IN), lambda i: (i, 0)),
            pl.BlockSpec((NC, BN, MSGW), lambda i: (0, i, 0)),
            full(IN, HID), full(1, HID), full(1, HID), full(1, HID),
            full(HID, OUT), full(1, OUT),
            full(OUT, HDIM), full(1, HDIM),
            full(HDIM, NA), full(1, NA),
        ],
        out_specs=pl.BlockSpec((BN, NA), lambda i: (i, 0)),
        out_shape=jax.ShapeDtypeStruct((N, NA), jnp.float32),
    )(x, agg, rootw, bconv, gamma, beta, wlin, blin, wq1, bq1, wq2, bq2)


def kernel(x, edge_index, edge_attr, W1, b1, root, bconv, gamma, beta,
           Wlin, blin, Wq1, bq1, Wq2, bq2):

    w1om = W1.reshape(IN, HID, EDIM).transpose(1, 0, 2).reshape(IN * HID, EDIM).T
    b1om = b1.reshape(IN, HID).T.reshape(1, IN * HID)
    w1om = w1om.astype(jnp.bfloat16)
    b1om = b1om.astype(jnp.bfloat16)
    col = jnp.arange(IN * HID, dtype=jnp.int32) // IN
    sel = (col[:, None] == jnp.arange(MSGW, dtype=jnp.int32)[None, :])
    sel = sel.astype(jnp.bfloat16)

    agg = jnp.zeros((NC, N, MSGW), jnp.float32)
    x_srcs = [_gather(k)(edge_index, x) for k in range(NCHUNK)]
    ea_bf = edge_attr.astype(jnp.bfloat16)
    msgs = [_edge_msg(k, x_srcs[k], ea_bf, w1om, b1om, sel)
            for k in range(NCHUNK)]
    for k in range(NCHUNK):
        agg = _scatter(k)(edge_index, msgs[k], agg)

    q = _node_head(x, agg, root, bconv.reshape(1, HID),
                   gamma.reshape(1, HID), beta.reshape(1, HID),
                   Wlin.T, blin.reshape(1, OUT), Wq1.T, bq1.reshape(1, HDIM),
                   Wq2.T, bq2.reshape(1, NA))
    return q
